# baseline (speedup 1.0000x reference)
import functools

import jax
import jax.numpy as jnp
from jax import lax
from jax.experimental import pallas as pl
from jax.experimental.pallas import tpu as pltpu
from jax.experimental.pallas import tpu_sc as plsc

_CHUNK = 128
_SUPER = 256
_NBUF = 2


@functools.lru_cache(maxsize=None)
def _build(B, D):
    info = plsc.get_sparse_core_info()
    nc, ns = info.num_cores, info.num_subcores
    nw = nc * ns
    per_w = B // nw
    n_super = per_w // _SUPER
    n_pair = n_super // _NBUF
    G = _SUPER // _CHUNK
    assert per_w * nw == B and n_super * _SUPER == per_w and n_pair * _NBUF == n_super
    mesh = plsc.VectorSubcoreMesh(core_axis_name="c", subcore_axis_name="s")

    @functools.partial(
        pl.kernel,
        mesh=mesh,
        out_type=jax.ShapeDtypeStruct((B, D), jnp.float32),
        scratch_types=[
            pltpu.VMEM((_NBUF, _SUPER), jnp.int32),
            pltpu.VMEM((_NBUF, _SUPER, D), jnp.float32),
            pltpu.SemaphoreType.DMA((_NBUF,)),
            pltpu.SemaphoreType.DMA((_NBUF,)),
            pltpu.SemaphoreType.DMA((_NBUF,)),
        ],
        compiler_params=pltpu.CompilerParams(use_tc_tiling_on_sc=False),
    )
    def gather_kernel(idx_hbm, table_hbm, out_hbm, idx_v, rows_v, isem, gsem, osem):
        wid = lax.axis_index("s") * nc + lax.axis_index("c")
        base = wid * per_w

        def idx_off(i):
            return pl.multiple_of(base + i * _SUPER, _SUPER)

        for b in range(_NBUF):
            pltpu.async_copy(
                idx_hbm.at[pl.ds(idx_off(b), _SUPER)], idx_v.at[b], isem.at[b])

        def pair(g, carry):
            for b in range(_NBUF):
                i = g * _NBUF + b

                @pl.when(g > 0)
                def _():
                    pltpu.make_async_copy(
                        rows_v.at[b], out_hbm.at[pl.ds(idx_off(0), _SUPER)],
                        osem.at[b]).wait()

                pltpu.make_async_copy(
                    idx_hbm.at[pl.ds(idx_off(0), _SUPER)], idx_v.at[b],
                    isem.at[b]).wait()

                for j in range(G):
                    pltpu.async_copy(
                        table_hbm.at[idx_v.at[b, pl.ds(j * _CHUNK, _CHUNK)]],
                        rows_v.at[b, pl.ds(j * _CHUNK, _CHUNK)],
                        gsem.at[b])

                pltpu.make_async_copy(
                    table_hbm.at[pl.ds(0, _SUPER)], rows_v.at[b],
                    gsem.at[b]).wait()

                @pl.when(g < n_pair - 1)
                def _():
                    pltpu.async_copy(
                        idx_hbm.at[pl.ds(idx_off(i + _NBUF), _SUPER)],
                        idx_v.at[b], isem.at[b])

                pltpu.async_copy(
                    rows_v.at[b], out_hbm.at[pl.ds(idx_off(i), _SUPER)],
                    osem.at[b])
            return carry

        lax.fori_loop(0, n_pair, pair, 0)
        for b in range(_NBUF):
            pltpu.make_async_copy(
                rows_v.at[b], out_hbm.at[pl.ds(idx_off(0), _SUPER)],
                osem.at[b]).wait()

    return gather_kernel


def _transpose_block(tt_ref, out_ref):
    d = tt_ref.shape[0]
    eye = jnp.eye(d, dtype=jnp.float32)
    t = jax.lax.dot_general(
        tt_ref[...], eye, (((0,), (0,)), ((), ())),
        preferred_element_type=jnp.float32,
        precision=jax.lax.Precision.HIGHEST)
    out_ref[...] = jnp.concatenate([t, t], axis=1)


@functools.lru_cache(maxsize=None)
def _build_transpose(V, D, DP):
    W = 2048
    return pl.pallas_call(
        _transpose_block,
        grid=((V + W - 1) // W,),
        in_specs=[pl.BlockSpec((D, W), lambda i: (0, i))],
        out_specs=pl.BlockSpec((W, DP), lambda i: (i, 0)),
        out_shape=jax.ShapeDtypeStruct((V, DP), jnp.float32),
    )


def kernel(x, table):
    B = x.shape[0] * x.shape[1]
    V, D = table.shape
    DP = 128
    idx = x.reshape(B).astype(jnp.int32)
    table_p = _build_transpose(V, D, DP)(table.T)
    out = _build(B, DP)(idx, table_p)
    return out[:, :D].reshape(x.shape + (D,))


# speedup vs baseline: 1.2985x; 1.2985x over previous
import functools

import jax
import jax.numpy as jnp
from jax import lax
from jax.experimental import pallas as pl
from jax.experimental.pallas import tpu as pltpu
from jax.experimental.pallas import tpu_sc as plsc

_CHUNK = 128
_SUPER = 512
_NBUF = 2
_LANES = 16


@functools.lru_cache(maxsize=None)
def _build(B, D, DP):
    info = plsc.get_sparse_core_info()
    nc, ns = info.num_cores, info.num_subcores
    nw = nc * ns
    per_w = B // nw
    n_super = per_w // _SUPER
    n_pair = n_super // _NBUF
    G = _SUPER // _CHUNK
    assert per_w * nw == B and n_super * _SUPER == per_w and n_pair * _NBUF == n_super
    mesh = plsc.VectorSubcoreMesh(core_axis_name="c", subcore_axis_name="s")

    @functools.partial(
        pl.kernel,
        mesh=mesh,
        out_type=jax.ShapeDtypeStruct((B, DP), jnp.float32),
        scratch_types=[
            pltpu.VMEM((_NBUF, _SUPER), jnp.int32),
            pltpu.VMEM((_NBUF, _SUPER, D), jnp.float32),
            pltpu.SemaphoreType.DMA((_NBUF,)),
            pltpu.SemaphoreType.DMA((_NBUF,)),
            pltpu.SemaphoreType.DMA((_NBUF,)),
        ],
        compiler_params=pltpu.CompilerParams(use_tc_tiling_on_sc=False),
    )
    def gather_kernel(idx_hbm, table_hbm, out_hbm, idx_v, rows_v, isem, gsem, osem):
        wid = lax.axis_index("s") * nc + lax.axis_index("c")
        base = wid * per_w

        def idx_off(i):
            return pl.multiple_of(base + i * _SUPER, _SUPER)

        for b in range(_NBUF):
            pltpu.async_copy(
                idx_hbm.at[pl.ds(idx_off(b), _SUPER)], idx_v.at[b], isem.at[b])

        def pair(g, carry):
            for b in range(_NBUF):
                i = g * _NBUF + b

                @pl.when(g > 0)
                def _():
                    pltpu.make_async_copy(
                        rows_v.at[b],
                        out_hbm.at[pl.ds(idx_off(0), _SUPER), pl.ds(0, D)],
                        osem.at[b]).wait()

                pltpu.make_async_copy(
                    idx_hbm.at[pl.ds(idx_off(0), _SUPER)], idx_v.at[b],
                    isem.at[b]).wait()

                for k in range(_SUPER // _LANES):
                    v = idx_v[b, pl.ds(k * _LANES, _LANES)]
                    idx_v[b, pl.ds(k * _LANES, _LANES)] = v + v

                for j in range(G):
                    pltpu.async_copy(
                        table_hbm.at[idx_v.at[b, pl.ds(j * _CHUNK, _CHUNK)]],
                        rows_v.at[b, pl.ds(j * _CHUNK, _CHUNK)],
                        gsem.at[b])

                pltpu.make_async_copy(
                    table_hbm.at[pl.ds(0, _SUPER)], rows_v.at[b],
                    gsem.at[b]).wait()

                @pl.when(g < n_pair - 1)
                def _():
                    pltpu.async_copy(
                        idx_hbm.at[pl.ds(idx_off(i + _NBUF), _SUPER)],
                        idx_v.at[b], isem.at[b])

                pltpu.async_copy(
                    rows_v.at[b],
                    out_hbm.at[pl.ds(idx_off(i), _SUPER), pl.ds(0, D)],
                    osem.at[b])
            return carry

        lax.fori_loop(0, n_pair, pair, 0)
        for b in range(_NBUF):
            pltpu.make_async_copy(
                rows_v.at[b],
                out_hbm.at[pl.ds(idx_off(0), _SUPER), pl.ds(0, D)],
                osem.at[b]).wait()

    return gather_kernel


def kernel(x, table):
    B = x.shape[0] * x.shape[1]
    V, D = table.shape
    DP = 128
    idx = x.reshape(B).astype(jnp.int32)
    table_p = jnp.pad(table, ((0, 0), (0, DP - D)))
    t64 = table_p.reshape(2 * V, D)
    out = _build(B, D, DP)(idx, t64)
    return out[:, :D].reshape(x.shape + (D,))
